# Initial kernel scaffold; baseline (speedup 1.0000x reference)
import jax
import jax.numpy as jnp
from jax.experimental import pallas as pl


def kernel(seed_sets, labels, entity_emb, W1, W2, q, sa_bias, out_bias):
    raise NotImplementedError("write your pallas kernel here")


# speedup vs baseline: 1.3985x; 1.3985x over previous
import functools
import math

import jax
import jax.numpy as jnp
import numpy as np
from jax import lax
from jax.experimental import pallas as pl
from jax.experimental.pallas import tpu as pltpu
from jax.experimental.pallas import tpu_sc as plsc

N_ENTITY = 100000
DIM = 128
B = 64
L = 64
BL = B * L
TILE = 8192
GRID = (N_ENTITY + TILE - 1) // TILE

NEG_INF = float("-inf")


def _pe_full_np():
    position = np.arange(0, L, dtype=np.float32)[:, None]
    div_term = np.exp(
        np.arange(0, DIM, 2).astype(np.float32) * (-math.log(10000.0) / DIM)
    )
    pe = np.zeros((L, DIM), dtype=np.float32)
    pe[:, 0::2] = np.sin(position * div_term) / 1000.0
    pe[:, 1::2] = np.cos(position * div_term) / 1000.0
    return np.tile(pe, (B, 1))


_PE_FULL = _pe_full_np()


def _make_sc_gather():
    info = plsc.get_sparse_core_info()
    nc, ns = info.num_cores, info.num_subcores
    nw = nc * ns
    bpw = BL // nw
    mesh = plsc.VectorSubcoreMesh(core_axis_name="c", subcore_axis_name="s")

    @functools.partial(
        pl.kernel,
        mesh=mesh,
        out_type=jax.ShapeDtypeStruct((BL, DIM), jnp.float32),
        scratch_types=[
            pltpu.VMEM((bpw,), jnp.int32),
            pltpu.VMEM((bpw, DIM), jnp.float32),
            pltpu.SemaphoreType.DMA,
        ],
    )
    def gather(table_hbm, idx_hbm, out_hbm, idx_v, rows_v, sem):
        wid = lax.axis_index("s") * nc + lax.axis_index("c")
        base = wid * bpw
        pltpu.sync_copy(idx_hbm.at[pl.ds(base, bpw)], idx_v)
        pltpu.async_copy(table_hbm.at[idx_v], rows_v, sem).wait()
        pltpu.sync_copy(rows_v, out_hbm.at[pl.ds(base, bpw)])

    return gather


_sc_gather_cache = []


def _get_sc_gather():
    if not _sc_gather_cache:
        _sc_gather_cache.append(_make_sc_gather())
    return _sc_gather_cache[0]


def _tc_body(
    v_ref, pe_ref, lab_ref, w1_ref, w2_ref, q_ref, sb_ref, emb_ref, ob_ref,
    scores_ref, loss_ref, u_s, m_s, s_s, ls_s,
):
    i = pl.program_id(0)

    @pl.when(i == 0)
    def _init():
        vp = v_ref[:] + pe_ref[:]
        qv = q_ref[:]
        qa = jnp.dot(qv, w1_ref[:], preferred_element_type=jnp.float32)
        qc = jnp.dot(qv, w2_ref[:], preferred_element_type=jnp.float32)
        att0 = jnp.sum(vp * qa, axis=1, keepdims=True)
        bi = lax.broadcasted_iota(jnp.int32, (B, BL), 0)
        ni = lax.broadcasted_iota(jnp.int32, (B, BL), 1)
        seg = (ni >> 6) == bi
        sb_mat = seg.astype(jnp.float32)
        sn_mat = (ni == bi * L + (L - 1)).astype(jnp.float32)
        vsum = jnp.dot(sb_mat, vp, preferred_element_type=jnp.float32)
        vn = jnp.dot(sn_mat, vp, preferred_element_type=jnp.float32)
        s0 = sb_ref[0, 0] * jnp.sum(qv)
        term = jnp.sum(vn * qc, axis=1, keepdims=True) + s0
        u1 = jnp.dot(sb_mat, att0 * vp, preferred_element_type=jnp.float32)
        u_s[:] = u1 + term * vsum
        m_s[:] = jnp.full((B, 1), NEG_INF, jnp.float32)
        s_s[:] = jnp.zeros((B, 1), jnp.float32)
        ls_s[:] = jnp.zeros((B, 1), jnp.float32)

    emb = emb_ref[:]
    st = lax.dot_general(
        u_s[:], emb, (((1,), (1,)), ((), ())),
        preferred_element_type=jnp.float32,
    )
    st = st + ob_ref[:]
    scores_ref[:] = st

    col = lax.broadcasted_iota(jnp.int32, (B, TILE), 1) + i * TILE
    stv = jnp.where(col < N_ENTITY, st, NEG_INF)
    mt = jnp.max(stv, axis=1, keepdims=True)
    m_old = m_s[:]
    m_new = jnp.maximum(m_old, mt)
    sexp = jnp.sum(jnp.exp(stv - m_new), axis=1, keepdims=True)
    s_s[:] = s_s[:] * jnp.exp(m_old - m_new) + sexp
    m_s[:] = m_new
    lmask = col == lab_ref[:]
    ls_s[:] = ls_s[:] + jnp.sum(jnp.where(lmask, st, 0.0), axis=1, keepdims=True)

    @pl.when(i == pl.num_programs(0) - 1)
    def _fini():
        lse = m_s[:] + jnp.log(s_s[:])
        loss_ref[:] = jnp.mean(lse - ls_s[:], axis=0, keepdims=True)


def _tc_scores_loss(v_flat, pe_full, labels2, w1, w2, q, sb2, entity_emb, ob2):
    const = lambda i: (0, 0)
    return pl.pallas_call(
        _tc_body,
        grid=(GRID,),
        in_specs=[
            pl.BlockSpec((BL, DIM), const),
            pl.BlockSpec((BL, DIM), const),
            pl.BlockSpec((B, 1), const),
            pl.BlockSpec((DIM, DIM), const),
            pl.BlockSpec((DIM, DIM), const),
            pl.BlockSpec((1, DIM), const),
            pl.BlockSpec((1, 1), const),
            pl.BlockSpec((TILE, DIM), lambda i: (i, 0)),
            pl.BlockSpec((1, TILE), lambda i: (0, i)),
        ],
        out_specs=[
            pl.BlockSpec((B, TILE), lambda i: (0, i)),
            pl.BlockSpec((1, 1), const),
        ],
        out_shape=[
            jax.ShapeDtypeStruct((B, N_ENTITY), jnp.float32),
            jax.ShapeDtypeStruct((1, 1), jnp.float32),
        ],
        scratch_shapes=[
            pltpu.VMEM((B, DIM), jnp.float32),
            pltpu.VMEM((B, 1), jnp.float32),
            pltpu.VMEM((B, 1), jnp.float32),
            pltpu.VMEM((B, 1), jnp.float32),
        ],
    )(v_flat, pe_full, labels2, w1, w2, q, sb2, entity_emb, ob2)


def kernel(seed_sets, labels, entity_emb, W1, W2, q, sa_bias, out_bias):
    seed_flat = seed_sets.reshape(-1).astype(jnp.int32)
    v_flat = _get_sc_gather()(entity_emb, seed_flat)
    pe_full = jnp.asarray(_PE_FULL)
    labels2 = labels.astype(jnp.int32).reshape(B, 1)
    sb2 = sa_bias.reshape(1, 1)
    ob2 = out_bias.reshape(1, N_ENTITY)
    scores, loss = _tc_scores_loss(
        v_flat, pe_full, labels2, W1, W2, q, sb2, entity_emb, ob2
    )
    return scores, loss.reshape(())
